# baseline (speedup 1.0000x reference)
import functools

import jax
import jax.numpy as jnp
import numpy as np
from jax.experimental import pallas as pl
from jax.experimental.pallas import tpu as pltpu

N = 8192
K = 32
D = 128
R = 256
C = 1024
NCHUNK = N // C
WCHUNK = 3
ALIGN = 512
IMAX = np.int32(0x7FFFFFFF)
_PREC = jax.lax.Precision.HIGHEST
_PREC_REF = jax.lax.Precision.DEFAULT


def _phases(i, hr, sqr, br, ha_ref, bcol_ref, keys_ref, keys16_ref, w_ref,
            b_ref, y_ref, out_ref, loss_ref, lo, nchunk, relu, last):
    ones = jnp.ones((1, D), jnp.float32)

    for ci in range(nchunk):
        off = pl.multiple_of(lo + ci * C, ALIGN)
        ha_c = ha_ref[pl.ds(off, C), :]
        g = jax.lax.dot_general(hr, ha_c, (((1,), (1,)), ((), ())),
                                preferred_element_type=jnp.float32,
                                precision=_PREC_REF)
        sqc = jax.lax.dot_general(ones, ha_c * ha_c, (((1,), (1,)), ((), ())),
                                  preferred_element_type=jnp.float32,
                                  precision=_PREC)
        dist = sqr + sqc - 2.0 * g
        u = jax.lax.bitcast_convert_type(dist, jnp.int32)
        key = u ^ ((u >> 31) & IMAX)
        bc = bcol_ref[0:1, pl.ds(off, C)]
        col_ids = off + jax.lax.broadcasted_iota(jnp.int32, (R, C), 1)
        row_ids = i * R + jax.lax.broadcasted_iota(jnp.int32, (R, C), 0)
        valid = (br == bc) & (col_ids != row_ids)
        key_m = jnp.where(valid, key, IMAX)
        keys_ref[:, ci * C:(ci + 1) * C] = key_m
        keys16_ref[:, ci * C:(ci + 1) * C] = (key_m >> 16).astype(jnp.int16)

    def count_le(t):
        c = jnp.zeros((R, 1), jnp.int32)
        for ci in range(nchunk):
            kc = keys_ref[:, ci * C:(ci + 1) * C]
            c = c + jnp.sum((kc <= t).astype(jnp.int32), axis=1,
                            keepdims=True)
        return c

    mc = keys_ref[:, 0:C]
    for ci in range(1, nchunk):
        mc = jnp.minimum(mc, keys_ref[:, ci * C:(ci + 1) * C])
    w = C
    while w > 32:
        w //= 2
        mc = jnp.minimum(mc[:, :w], mc[:, w:2 * w])
    ub16 = jnp.max(mc, axis=1, keepdims=True) >> 16
    lb16 = jnp.min(mc, axis=1, keepdims=True) >> 16

    def count16(t16i):
        t16 = t16i.astype(jnp.int16)
        c = jnp.zeros((R, 1), jnp.int32)
        for ci in range(nchunk):
            kc = keys16_ref[:, ci * C:(ci + 1) * C]
            m = (kc <= t16).astype(jnp.int16)
            w2 = C
            while w2 > 128:
                w2 //= 2
                m = m[:, :w2] + m[:, w2:2 * w2]
            c = c + jnp.sum(m.astype(jnp.int32), axis=1, keepdims=True)
        return c

    def co_cond(carry):
        it, lo_, hi_, _, res = carry
        active = jnp.logical_and(res == 0, hi_ - lo_ > 1)
        return jnp.logical_and(it < 16, jnp.sum(active.astype(jnp.int32)) > 0)

    def co_body(carry):
        it, lo_, hi_, v_, res = carry
        mid = lo_ + ((hi_ - lo_) >> 1)
        c = count16(mid)
        hit = jnp.logical_and(c == K, res == 0)
        v_ = jnp.where(hit, (mid << 16) | jnp.int32(0xFFFF), v_)
        res = jnp.where(hit, jnp.int32(1), res)
        lt = c < K
        lo_ = jnp.where(lt, mid, lo_)
        hi_ = jnp.where(lt, hi_, mid)
        return it + 1, lo_, hi_, v_, res

    zero = jnp.zeros((R, 1), jnp.int32)
    _, lo16, hi16, v_c, res_c = jax.lax.while_loop(
        co_cond, co_body, (jnp.int32(0), lb16 - 1, ub16, zero, zero))

    def bi_round(lo_, hi_, v_, res):
        d = hi_ - lo_
        mid = lo_ + ((d >> 1) & IMAX)
        c = count_le(mid)
        hit = jnp.logical_and(c == K, res == 0)
        v_ = jnp.where(hit, mid, v_)
        res = jnp.where(hit, jnp.int32(1), res)
        lt = c < K
        lo_ = jnp.where(lt, mid, lo_)
        hi_ = jnp.where(lt, hi_, mid)
        return lo_, hi_, v_, res

    def bi_cond(carry):
        it, _, _, _, res = carry
        return jnp.logical_and(it < 17, jnp.sum(res) < R)

    def bi_body(carry):
        it, lo_, hi_, v_, res = carry
        lo_, hi_, v_, res = bi_round(lo_, hi_, v_, res)
        lo_, hi_, v_, res = bi_round(lo_, hi_, v_, res)
        return it + 1, lo_, hi_, v_, res

    lo_i = (lo16 << 16) | jnp.int32(0xFFFF)
    hi_i = (hi16 << 16) | jnp.int32(0xFFFF)
    _, _, hi_f, v, res_f = jax.lax.while_loop(
        bi_cond, bi_body, (jnp.int32(0), lo_i, hi_i, v_c, res_c))
    v = jnp.where(res_f == 1, v, hi_f)

    acc = jnp.zeros((R, D), jnp.float32)
    cnt = jnp.zeros((R, 1), jnp.float32)
    for ci in range(nchunk):
        kc = keys_ref[:, ci * C:(ci + 1) * C]
        mc = ((kc <= v) & (kc != IMAX)).astype(jnp.float32)
        cnt = cnt + jnp.sum(mc, axis=1, keepdims=True)
        ha_c = ha_ref[pl.ds(pl.multiple_of(lo + ci * C, ALIGN), C), :]
        acc = acc + jax.lax.dot_general(mc, ha_c, (((1,), (0,)), ((), ())),
                                        preferred_element_type=jnp.float32,
                                        precision=_PREC)
    agg = acc / jnp.maximum(cnt, 1.0)

    out = jax.lax.dot_general(agg, w_ref[...], (((1,), (0,)), ((), ())),
                              preferred_element_type=jnp.float32,
                              precision=_PREC_REF) + b_ref[...]
    if relu:
        out = jnp.maximum(out, 0.0)
    out_ref[...] = out

    if last:
        yb = y_ref[...]
        d2 = (out - yb) ** 2
        part = jnp.sum(jnp.sum(d2, axis=1, keepdims=True), axis=0,
                       keepdims=True)
        prev = jnp.where(i == 0, jnp.zeros((1, 1), jnp.float32),
                         loss_ref[...])
        total = prev + part
        loss_ref[...] = jnp.where(i == pl.num_programs(0) - 1,
                                  total / jnp.float32(N * 3), total)


def _layer_kernel(lo_ref, span_ref, hr_ref, ha_ref, brow_ref, bcol_ref,
                  w_ref, b_ref, y_ref, out_ref, loss_ref, keys_ref,
                  keys16_ref, *, relu, last):
    i = pl.program_id(0)
    hr = hr_ref[...]
    sqr = jnp.sum(hr * hr, axis=1, keepdims=True)
    br = brow_ref[...]
    body = functools.partial(_phases, i, hr, sqr, br, ha_ref, bcol_ref,
                             keys_ref, keys16_ref, w_ref, b_ref, y_ref,
                             out_ref, loss_ref, relu=relu, last=last)
    fits = span_ref[i] <= WCHUNK * C

    @pl.when(fits)
    def _windowed():
        body(lo=lo_ref[i], nchunk=WCHUNK)

    @pl.when(jnp.logical_not(fits))
    def _full():
        body(lo=jnp.int32(0), nchunk=NCHUNK)


def _layer(h, brow, bcol, lo_al, span_al, w, b, y, relu, last):
    kern = functools.partial(_layer_kernel, relu=relu, last=last)
    grid_spec = pltpu.PrefetchScalarGridSpec(
        num_scalar_prefetch=2,
        grid=(N // R,),
        in_specs=[
            pl.BlockSpec((R, D), lambda i, *_: (i, 0)),
            pl.BlockSpec((N, D), lambda i, *_: (0, 0)),
            pl.BlockSpec((R, 1), lambda i, *_: (i, 0)),
            pl.BlockSpec((1, N), lambda i, *_: (0, 0)),
            pl.BlockSpec((D, D), lambda i, *_: (0, 0)),
            pl.BlockSpec((1, D), lambda i, *_: (0, 0)),
            pl.BlockSpec((R, D), lambda i, *_: (i, 0)),
        ],
        out_specs=[
            pl.BlockSpec((R, D), lambda i, *_: (i, 0)),
            pl.BlockSpec((1, 1), lambda i, *_: (0, 0)),
        ],
        scratch_shapes=[pltpu.VMEM((R, N), jnp.int32),
                        pltpu.VMEM((R, N), jnp.int16)],
    )
    out_shape = [
        jax.ShapeDtypeStruct((N, D), jnp.float32),
        jax.ShapeDtypeStruct((1, 1), jnp.float32),
    ]
    return pl.pallas_call(kern, grid_spec=grid_spec, out_shape=out_shape)(
        lo_al, span_al, h, h, brow, bcol, w, b, y)


def _pad_w(w):
    return jnp.pad(w, ((0, D - w.shape[0]), (0, D - w.shape[1])))


def _pad_b(b):
    return jnp.pad(b, (0, D - b.shape[0])).reshape(1, D)


def kernel(x, batch, y, W1, b1, W2, b2, W3, b3):
    h = jnp.pad(x, ((0, 0), (0, D - x.shape[1])))
    yp = jnp.pad(y, ((0, 0), (0, D - y.shape[1])))
    brow = batch.reshape(N, 1)
    bcol = batch.reshape(1, N)
    r0 = jnp.arange(0, N, R)
    b0 = batch[r0]
    b1_ = batch[r0 + R - 1]
    lo = jnp.searchsorted(batch, b0, side="left").astype(jnp.int32)
    hi = jnp.searchsorted(batch, b1_, side="right").astype(jnp.int32)
    lo_al = (lo // ALIGN) * ALIGN
    lo_al = jnp.minimum(lo_al, N - WCHUNK * C)
    span_al = hi - lo_al
    h1, _ = _layer(h, brow, bcol, lo_al, span_al, _pad_w(W1), _pad_b(b1), yp,
                   True, False)
    h2, _ = _layer(h1, brow, bcol, lo_al, span_al, _pad_w(W2), _pad_b(b2), yp,
                   True, False)
    h3, loss = _layer(h2, brow, bcol, lo_al, span_al, _pad_w(W3), _pad_b(b3),
                      yp, False, True)
    return h3[:, :3], loss[0, 0]


# speedup vs baseline: 1.3447x; 1.3447x over previous
import functools

import jax
import jax.numpy as jnp
import numpy as np
from jax.experimental import pallas as pl
from jax.experimental.pallas import tpu as pltpu

N = 8192
K = 32
D = 128
R = 256
C = 1024
NCHUNK = N // C
WCHUNK = 3
ALIGN = 512
IMAX = np.int32(0x7FFFFFFF)
def _dot3(mb, a, dn):
    a1 = a.astype(jnp.bfloat16)
    r1 = a - a1.astype(jnp.float32)
    a2 = r1.astype(jnp.bfloat16)
    r2 = r1 - a2.astype(jnp.float32)
    a3 = r2.astype(jnp.bfloat16)

    def d(x):
        return jax.lax.dot_general(mb, x, dn,
                                   preferred_element_type=jnp.float32)

    return d(a1) + d(a2) + d(a3)
_PREC_REF = jax.lax.Precision.DEFAULT


def _phases(i, hr, sqr, br, ha_ref, bcol_ref, keys_ref, w_ref,
            b_ref, y_ref, out_ref, loss_ref, lo, nchunk, relu, last):
    ones = jnp.ones((1, D), jnp.bfloat16)

    for ci in range(nchunk):
        off = pl.multiple_of(lo + ci * C, ALIGN)
        ha_c = ha_ref[pl.ds(off, C), :]
        g = jax.lax.dot_general(hr, ha_c, (((1,), (1,)), ((), ())),
                                preferred_element_type=jnp.float32,
                                precision=_PREC_REF)
        sqc = _dot3(ones, ha_c * ha_c, (((1,), (1,)), ((), ())))
        dist = sqr + sqc - 2.0 * g
        u = jax.lax.bitcast_convert_type(dist, jnp.int32)
        key = u ^ ((u >> 31) & IMAX)
        bc = bcol_ref[0:1, pl.ds(off, C)]
        col_ids = off + jax.lax.broadcasted_iota(jnp.int32, (R, C), 1)
        row_ids = i * R + jax.lax.broadcasted_iota(jnp.int32, (R, C), 0)
        valid = (br == bc) & (col_ids != row_ids)
        keys_ref[:, ci * C:(ci + 1) * C] = jnp.where(valid, key, IMAX)

    def count_le(t):
        c = jnp.zeros((R, 1), jnp.int32)
        for ci in range(nchunk):
            kc = keys_ref[:, ci * C:(ci + 1) * C]
            c = c + jnp.sum((kc <= t).astype(jnp.int32), axis=1,
                            keepdims=True)
        return c

    mc = keys_ref[:, 0:C]
    for ci in range(1, nchunk):
        mc = jnp.minimum(mc, keys_ref[:, ci * C:(ci + 1) * C])
    w = C
    while w > 32:
        w //= 2
        mc = jnp.minimum(mc[:, :w], mc[:, w:2 * w])
    ub = jnp.max(mc, axis=1, keepdims=True)
    lb = jnp.min(mc, axis=1, keepdims=True)

    def bi_round(lo_, hi_, v_, res):
        d = hi_ - lo_
        mid = lo_ + ((d >> 1) & IMAX)
        c = count_le(mid)
        hit = jnp.logical_and(c == K, res == 0)
        v_ = jnp.where(hit, mid, v_)
        res = jnp.where(hit, jnp.int32(1), res)
        lt = c < K
        lo_ = jnp.where(lt, mid, lo_)
        hi_ = jnp.where(lt, hi_, mid)
        return lo_, hi_, v_, res

    def bi_cond(carry):
        it, _, _, _, res = carry
        return jnp.logical_and(it < 17, jnp.sum(res) < R)

    def bi_body(carry):
        it, lo_, hi_, v_, res = carry
        lo_, hi_, v_, res = bi_round(lo_, hi_, v_, res)
        lo_, hi_, v_, res = bi_round(lo_, hi_, v_, res)
        return it + 1, lo_, hi_, v_, res

    zero = jnp.zeros((R, 1), jnp.int32)
    _, _, hi_f, v, res_f = jax.lax.while_loop(
        bi_cond, bi_body, (jnp.int32(0), lb - 1, ub, zero, zero))
    v = jnp.where(res_f == 1, v, hi_f)

    acc = jnp.zeros((R, D), jnp.float32)
    cnt = jnp.zeros((R, 1), jnp.float32)
    for ci in range(nchunk):
        kc = keys_ref[:, ci * C:(ci + 1) * C]
        mc = ((kc <= v) & (kc != IMAX)).astype(jnp.float32)
        cnt = cnt + jnp.sum(mc, axis=1, keepdims=True)
        ha_c = ha_ref[pl.ds(pl.multiple_of(lo + ci * C, ALIGN), C), :]
        acc = acc + _dot3(mc.astype(jnp.bfloat16), ha_c,
                          (((1,), (0,)), ((), ())))
    agg = acc / jnp.maximum(cnt, 1.0)

    out = jax.lax.dot_general(agg, w_ref[...], (((1,), (0,)), ((), ())),
                              preferred_element_type=jnp.float32,
                              precision=_PREC_REF) + b_ref[...]
    if relu:
        out = jnp.maximum(out, 0.0)
    out_ref[...] = out

    if last:
        yb = y_ref[...]
        d2 = (out - yb) ** 2
        part = jnp.sum(jnp.sum(d2, axis=1, keepdims=True), axis=0,
                       keepdims=True)
        prev = jnp.where(i == 0, jnp.zeros((1, 1), jnp.float32),
                         loss_ref[...])
        total = prev + part
        loss_ref[...] = jnp.where(i == pl.num_programs(0) - 1,
                                  total / jnp.float32(N * 3), total)


def _layer_kernel(lo_ref, span_ref, hr_ref, ha_ref, brow_ref, bcol_ref,
                  w_ref, b_ref, y_ref, out_ref, loss_ref, keys_ref, *,
                  relu, last):
    i = pl.program_id(0)
    hr = hr_ref[...]
    sqr = jnp.sum(hr * hr, axis=1, keepdims=True)
    br = brow_ref[...]
    body = functools.partial(_phases, i, hr, sqr, br, ha_ref, bcol_ref,
                             keys_ref, w_ref, b_ref, y_ref,
                             out_ref, loss_ref, relu=relu, last=last)
    fits = span_ref[i] <= WCHUNK * C

    @pl.when(fits)
    def _windowed():
        body(lo=lo_ref[i], nchunk=WCHUNK)

    @pl.when(jnp.logical_not(fits))
    def _full():
        body(lo=jnp.int32(0), nchunk=NCHUNK)


def _layer(h, brow, bcol, lo_al, span_al, w, b, y, relu, last):
    kern = functools.partial(_layer_kernel, relu=relu, last=last)
    grid_spec = pltpu.PrefetchScalarGridSpec(
        num_scalar_prefetch=2,
        grid=(N // R,),
        in_specs=[
            pl.BlockSpec((R, D), lambda i, *_: (i, 0)),
            pl.BlockSpec((N, D), lambda i, *_: (0, 0)),
            pl.BlockSpec((R, 1), lambda i, *_: (i, 0)),
            pl.BlockSpec((1, N), lambda i, *_: (0, 0)),
            pl.BlockSpec((D, D), lambda i, *_: (0, 0)),
            pl.BlockSpec((1, D), lambda i, *_: (0, 0)),
            pl.BlockSpec((R, D), lambda i, *_: (i, 0)),
        ],
        out_specs=[
            pl.BlockSpec((R, D), lambda i, *_: (i, 0)),
            pl.BlockSpec((1, 1), lambda i, *_: (0, 0)),
        ],
        scratch_shapes=[pltpu.VMEM((R, N), jnp.int32)],
    )
    out_shape = [
        jax.ShapeDtypeStruct((N, D), jnp.float32),
        jax.ShapeDtypeStruct((1, 1), jnp.float32),
    ]
    return pl.pallas_call(kern, grid_spec=grid_spec, out_shape=out_shape)(
        lo_al, span_al, h, h, brow, bcol, w, b, y)


def _pad_w(w):
    return jnp.pad(w, ((0, D - w.shape[0]), (0, D - w.shape[1])))


def _pad_b(b):
    return jnp.pad(b, (0, D - b.shape[0])).reshape(1, D)


def kernel(x, batch, y, W1, b1, W2, b2, W3, b3):
    h = jnp.pad(x, ((0, 0), (0, D - x.shape[1])))
    yp = jnp.pad(y, ((0, 0), (0, D - y.shape[1])))
    brow = batch.reshape(N, 1)
    bcol = batch.reshape(1, N)
    r0 = jnp.arange(0, N, R)
    b0 = batch[r0]
    b1_ = batch[r0 + R - 1]
    lo = jnp.searchsorted(batch, b0, side="left").astype(jnp.int32)
    hi = jnp.searchsorted(batch, b1_, side="right").astype(jnp.int32)
    lo_al = (lo // ALIGN) * ALIGN
    lo_al = jnp.minimum(lo_al, N - WCHUNK * C)
    span_al = hi - lo_al
    h1, _ = _layer(h, brow, bcol, lo_al, span_al, _pad_w(W1), _pad_b(b1), yp,
                   True, False)
    h2, _ = _layer(h1, brow, bcol, lo_al, span_al, _pad_w(W2), _pad_b(b2), yp,
                   True, False)
    h3, loss = _layer(h2, brow, bcol, lo_al, span_al, _pad_w(W3), _pad_b(b3),
                      yp, False, True)
    return h3[:, :3], loss[0, 0]


# speedup vs baseline: 1.7293x; 1.2860x over previous
import functools

import jax
import jax.numpy as jnp
import numpy as np
from jax.experimental import pallas as pl
from jax.experimental.pallas import tpu as pltpu

N = 8192
K = 32
D = 128
R = 256
C = 1024
NCHUNK = N // C
WCHUNK = 3
ALIGN = 128
IMAX = np.int32(0x7FFFFFFF)
def _dot3(mb, a, dn):
    a1 = a.astype(jnp.bfloat16)
    r1 = a - a1.astype(jnp.float32)
    a2 = r1.astype(jnp.bfloat16)
    r2 = r1 - a2.astype(jnp.float32)
    a3 = r2.astype(jnp.bfloat16)

    def d(x):
        return jax.lax.dot_general(mb, x, dn,
                                   preferred_element_type=jnp.float32)

    return d(a1) + d(a2) + d(a3)
_PREC_REF = jax.lax.Precision.DEFAULT


def _phases(i, hr, sqr, br, ha_ref, bcol_ref, keys_ref, w_ref,
            b_ref, y_ref, out_ref, loss_ref, lo, nchunk, relu, last):
    ones = jnp.ones((1, D), jnp.bfloat16)

    for ci in range(nchunk):
        off = pl.multiple_of(lo + ci * C, ALIGN)
        ha_c = ha_ref[pl.ds(off, C), :]
        g = jax.lax.dot_general(hr, ha_c, (((1,), (1,)), ((), ())),
                                preferred_element_type=jnp.float32,
                                precision=_PREC_REF)
        sqc = _dot3(ones, ha_c * ha_c, (((1,), (1,)), ((), ())))
        dist = sqr + sqc - 2.0 * g
        u = jax.lax.bitcast_convert_type(dist, jnp.int32)
        key = u ^ ((u >> 31) & IMAX)
        bc = bcol_ref[0:1, pl.ds(off, C)]
        col_ids = off + jax.lax.broadcasted_iota(jnp.int32, (R, C), 1)
        row_ids = i * R + jax.lax.broadcasted_iota(jnp.int32, (R, C), 0)
        valid = (br == bc) & (col_ids != row_ids)
        keys_ref[:, ci * C:(ci + 1) * C] = jnp.where(valid, key, IMAX)

    def count_le(t):
        c = jnp.zeros((R, 1), jnp.int32)
        for ci in range(nchunk):
            kc = keys_ref[:, ci * C:(ci + 1) * C]
            c = c + jnp.sum((kc <= t).astype(jnp.int32), axis=1,
                            keepdims=True)
        return c

    mc = keys_ref[:, 0:C]
    for ci in range(1, nchunk):
        mc = jnp.minimum(mc, keys_ref[:, ci * C:(ci + 1) * C])
    w = C
    while w > 32:
        w //= 2
        mc = jnp.minimum(mc[:, :w], mc[:, w:2 * w])
    ub = jnp.max(mc, axis=1, keepdims=True)
    lb = jnp.min(mc, axis=1, keepdims=True)

    def bi_round(lo_, hi_, v_, res):
        d = hi_ - lo_
        mid = lo_ + ((d >> 1) & IMAX)
        c = count_le(mid)
        hit = jnp.logical_and(c == K, res == 0)
        v_ = jnp.where(hit, mid, v_)
        res = jnp.where(hit, jnp.int32(1), res)
        lt = c < K
        lo_ = jnp.where(lt, mid, lo_)
        hi_ = jnp.where(lt, hi_, mid)
        return lo_, hi_, v_, res

    def bi_cond(carry):
        it, _, _, _, res = carry
        return jnp.logical_and(it < 17, jnp.sum(res) < R)

    def bi_body(carry):
        it, lo_, hi_, v_, res = carry
        lo_, hi_, v_, res = bi_round(lo_, hi_, v_, res)
        lo_, hi_, v_, res = bi_round(lo_, hi_, v_, res)
        return it + 1, lo_, hi_, v_, res

    zero = jnp.zeros((R, 1), jnp.int32)
    _, _, hi_f, v, res_f = jax.lax.while_loop(
        bi_cond, bi_body, (jnp.int32(0), lb - 1, ub, zero, zero))
    v = jnp.where(res_f == 1, v, hi_f)

    acc = jnp.zeros((R, D), jnp.float32)
    cnt = jnp.zeros((R, 1), jnp.float32)
    for ci in range(nchunk):
        kc = keys_ref[:, ci * C:(ci + 1) * C]
        mc = ((kc <= v) & (kc != IMAX)).astype(jnp.float32)
        cnt = cnt + jnp.sum(mc, axis=1, keepdims=True)
        ha_c = ha_ref[pl.ds(pl.multiple_of(lo + ci * C, ALIGN), C), :]
        acc = acc + _dot3(mc.astype(jnp.bfloat16), ha_c,
                          (((1,), (0,)), ((), ())))
    agg = acc / jnp.maximum(cnt, 1.0)

    out = jax.lax.dot_general(agg, w_ref[...], (((1,), (0,)), ((), ())),
                              preferred_element_type=jnp.float32,
                              precision=_PREC_REF) + b_ref[...]
    if relu:
        out = jnp.maximum(out, 0.0)
    out_ref[...] = out

    if last:
        yb = y_ref[...]
        d2 = (out - yb) ** 2
        part = jnp.sum(jnp.sum(d2, axis=1, keepdims=True), axis=0,
                       keepdims=True)
        prev = jnp.where(i == 0, jnp.zeros((1, 1), jnp.float32),
                         loss_ref[...])
        total = prev + part
        loss_ref[...] = jnp.where(i == pl.num_programs(0) - 1,
                                  total / jnp.float32(N * 3), total)


def _layer_kernel(lo_ref, hi_ref, hr_ref, ha_ref, brow_ref, bcol_ref,
                  w_ref, b_ref, y_ref, out_ref, loss_ref, keys_ref, *,
                  relu, last):
    i = pl.program_id(0)
    hr = hr_ref[...]
    sqr = jnp.sum(hr * hr, axis=1, keepdims=True)
    br = brow_ref[...]
    body = functools.partial(_phases, i, hr, sqr, br, ha_ref, bcol_ref,
                             keys_ref, w_ref, b_ref, y_ref,
                             out_ref, loss_ref, relu=relu, last=last)
    lo_a = lo_ref[i]
    hi = hi_ref[i]
    lo2 = jnp.minimum(lo_a, jnp.int32(N - 2 * C))
    fits2 = hi - lo2 <= 2 * C
    lo3 = jnp.minimum(lo_a, jnp.int32(N - 3 * C))
    fits3 = hi - lo3 <= 3 * C

    @pl.when(fits2)
    def _win2():
        body(lo=lo2, nchunk=2)

    @pl.when(jnp.logical_and(jnp.logical_not(fits2), fits3))
    def _win3():
        body(lo=lo3, nchunk=3)

    @pl.when(jnp.logical_not(fits3))
    def _full():
        body(lo=jnp.int32(0), nchunk=NCHUNK)


def _layer(h, brow, bcol, lo_al, hi, w, b, y, relu, last):
    kern = functools.partial(_layer_kernel, relu=relu, last=last)
    grid_spec = pltpu.PrefetchScalarGridSpec(
        num_scalar_prefetch=2,
        grid=(N // R,),
        in_specs=[
            pl.BlockSpec((R, D), lambda i, *_: (i, 0)),
            pl.BlockSpec((N, D), lambda i, *_: (0, 0)),
            pl.BlockSpec((R, 1), lambda i, *_: (i, 0)),
            pl.BlockSpec((1, N), lambda i, *_: (0, 0)),
            pl.BlockSpec((D, D), lambda i, *_: (0, 0)),
            pl.BlockSpec((1, D), lambda i, *_: (0, 0)),
            pl.BlockSpec((R, D), lambda i, *_: (i, 0)),
        ],
        out_specs=[
            pl.BlockSpec((R, D), lambda i, *_: (i, 0)),
            pl.BlockSpec((1, 1), lambda i, *_: (0, 0)),
        ],
        scratch_shapes=[pltpu.VMEM((R, N), jnp.int32)],
    )
    out_shape = [
        jax.ShapeDtypeStruct((N, D), jnp.float32),
        jax.ShapeDtypeStruct((1, 1), jnp.float32),
    ]
    return pl.pallas_call(kern, grid_spec=grid_spec, out_shape=out_shape)(
        lo_al, hi, h, h, brow, bcol, w, b, y)


def _pad_w(w):
    return jnp.pad(w, ((0, D - w.shape[0]), (0, D - w.shape[1])))


def _pad_b(b):
    return jnp.pad(b, (0, D - b.shape[0])).reshape(1, D)


def kernel(x, batch, y, W1, b1, W2, b2, W3, b3):
    h = jnp.pad(x, ((0, 0), (0, D - x.shape[1])))
    yp = jnp.pad(y, ((0, 0), (0, D - y.shape[1])))
    brow = batch.reshape(N, 1)
    bcol = batch.reshape(1, N)
    r0 = jnp.arange(0, N, R)
    b0 = batch[r0]
    b1_ = batch[r0 + R - 1]
    lo = jnp.searchsorted(batch, b0, side="left").astype(jnp.int32)
    hi = jnp.searchsorted(batch, b1_, side="right").astype(jnp.int32)
    lo_al = (lo // ALIGN) * ALIGN
    h1, _ = _layer(h, brow, bcol, lo_al, hi, _pad_w(W1), _pad_b(b1), yp,
                   True, False)
    h2, _ = _layer(h1, brow, bcol, lo_al, hi, _pad_w(W2), _pad_b(b2), yp,
                   True, False)
    h3, loss = _layer(h2, brow, bcol, lo_al, hi, _pad_w(W3), _pad_b(b3),
                      yp, False, True)
    return h3[:, :3], loss[0, 0]


# speedup vs baseline: 1.7714x; 1.0243x over previous
import functools

import jax
import jax.numpy as jnp
import numpy as np
from jax.experimental import pallas as pl
from jax.experimental.pallas import tpu as pltpu

N = 8192
K = 32
D = 128
R = 256
C = 1024
NCHUNK = N // C
WCHUNK = 3
ALIGN = 128
IMAX = np.int32(0x7FFFFFFF)
def _dot3(mb, a, dn):
    a1 = a.astype(jnp.bfloat16)
    r1 = a - a1.astype(jnp.float32)
    a2 = r1.astype(jnp.bfloat16)
    r2 = r1 - a2.astype(jnp.float32)
    a3 = r2.astype(jnp.bfloat16)

    def d(x):
        return jax.lax.dot_general(mb, x, dn,
                                   preferred_element_type=jnp.float32)

    return d(a1) + d(a2) + d(a3)
_PREC_REF = jax.lax.Precision.DEFAULT


def _phases(i, hr, sqr, br, ha_ref, bcol_ref, keys_ref, w_ref,
            b_ref, y_ref, out_ref, loss_ref, lo, nchunk, csize, relu, last):
    ones = jnp.ones((1, D), jnp.bfloat16)

    for ci in range(nchunk):
        off = pl.multiple_of(lo + ci * csize, ALIGN)
        ha_c = ha_ref[pl.ds(off, csize), :]
        g = jax.lax.dot_general(hr, ha_c, (((1,), (1,)), ((), ())),
                                preferred_element_type=jnp.float32,
                                precision=_PREC_REF)
        sqc = _dot3(ones, ha_c * ha_c, (((1,), (1,)), ((), ())))
        dist = sqr + sqc - 2.0 * g
        u = jax.lax.bitcast_convert_type(dist, jnp.int32)
        key = u ^ ((u >> 31) & IMAX)
        bc = bcol_ref[0:1, pl.ds(off, csize)]
        col_ids = off + jax.lax.broadcasted_iota(jnp.int32, (R, csize), 1)
        row_ids = i * R + jax.lax.broadcasted_iota(jnp.int32, (R, csize), 0)
        valid = (br == bc) & (col_ids != row_ids)
        keys_ref[:, ci * csize:(ci + 1) * csize] = jnp.where(valid, key, IMAX)

    def count_le(t):
        c = jnp.zeros((R, 1), jnp.int32)
        for ci in range(nchunk):
            kc = keys_ref[:, ci * csize:(ci + 1) * csize]
            c = c + jnp.sum((kc <= t).astype(jnp.int32), axis=1,
                            keepdims=True)
        return c

    mc = keys_ref[:, 0:csize]
    for ci in range(1, nchunk):
        mc = jnp.minimum(mc, keys_ref[:, ci * csize:(ci + 1) * csize])
    w = csize
    while w > 32:
        w //= 2
        mc = jnp.minimum(mc[:, :w], mc[:, w:2 * w])
    ub = jnp.max(mc, axis=1, keepdims=True)
    lb = jnp.min(mc, axis=1, keepdims=True)

    def bi_round(lo_, hi_, v_, res):
        d = hi_ - lo_
        mid = lo_ + ((d >> 1) & IMAX)
        c = count_le(mid)
        hit = jnp.logical_and(c == K, res == 0)
        v_ = jnp.where(hit, mid, v_)
        res = jnp.where(hit, jnp.int32(1), res)
        lt = c < K
        lo_ = jnp.where(lt, mid, lo_)
        hi_ = jnp.where(lt, hi_, mid)
        return lo_, hi_, v_, res

    def bi_cond(carry):
        it, _, _, _, res = carry
        return jnp.logical_and(it < 17, jnp.sum(res) < R)

    def bi_body(carry):
        it, lo_, hi_, v_, res = carry
        lo_, hi_, v_, res = bi_round(lo_, hi_, v_, res)
        lo_, hi_, v_, res = bi_round(lo_, hi_, v_, res)
        return it + 1, lo_, hi_, v_, res

    zero = jnp.zeros((R, 1), jnp.int32)
    _, _, hi_f, v, res_f = jax.lax.while_loop(
        bi_cond, bi_body, (jnp.int32(0), lb - 1, ub, zero, zero))
    v = jnp.where(res_f == 1, v, hi_f)

    acc = jnp.zeros((R, D), jnp.float32)
    cnt = jnp.zeros((R, 1), jnp.float32)
    for ci in range(nchunk):
        kc = keys_ref[:, ci * csize:(ci + 1) * csize]
        mc = ((kc <= v) & (kc != IMAX)).astype(jnp.float32)
        cnt = cnt + jnp.sum(mc, axis=1, keepdims=True)
        ha_c = ha_ref[pl.ds(pl.multiple_of(lo + ci * csize, ALIGN), csize), :]
        acc = acc + _dot3(mc.astype(jnp.bfloat16), ha_c,
                          (((1,), (0,)), ((), ())))
    agg = acc / jnp.maximum(cnt, 1.0)

    out = jax.lax.dot_general(agg, w_ref[...], (((1,), (0,)), ((), ())),
                              preferred_element_type=jnp.float32,
                              precision=_PREC_REF) + b_ref[...]
    if relu:
        out = jnp.maximum(out, 0.0)
    out_ref[...] = out

    if last:
        yb = y_ref[...]
        d2 = (out - yb) ** 2
        part = jnp.sum(jnp.sum(d2, axis=1, keepdims=True), axis=0,
                       keepdims=True)
        prev = jnp.where(i == 0, jnp.zeros((1, 1), jnp.float32),
                         loss_ref[...])
        total = prev + part
        loss_ref[...] = jnp.where(i == pl.num_programs(0) - 1,
                                  total / jnp.float32(N * 3), total)


def _layer_kernel(lo_ref, hi_ref, hr_ref, ha_ref, brow_ref, bcol_ref,
                  w_ref, b_ref, y_ref, out_ref, loss_ref, keys_ref, *,
                  relu, last):
    i = pl.program_id(0)
    hr = hr_ref[...]
    sqr = jnp.sum(hr * hr, axis=1, keepdims=True)
    br = brow_ref[...]
    body = functools.partial(_phases, i, hr, sqr, br, ha_ref, bcol_ref,
                             keys_ref, w_ref, b_ref, y_ref,
                             out_ref, loss_ref, relu=relu, last=last)
    lo_a = lo_ref[i]
    hi = hi_ref[i]
    lo1 = jnp.minimum(lo_a, jnp.int32(N - 1536))
    fits1 = hi - lo1 <= 1536
    lo3 = jnp.minimum(lo_a, jnp.int32(N - 3 * C))
    fits3 = hi - lo3 <= 3 * C

    @pl.when(fits1)
    def _win1():
        body(lo=lo1, nchunk=3, csize=512)

    @pl.when(jnp.logical_and(jnp.logical_not(fits1), fits3))
    def _win3():
        body(lo=lo3, nchunk=3, csize=C)

    @pl.when(jnp.logical_not(fits3))
    def _full():
        body(lo=jnp.int32(0), nchunk=NCHUNK, csize=C)


def _layer(h, brow, bcol, lo_al, hi, w, b, y, relu, last):
    kern = functools.partial(_layer_kernel, relu=relu, last=last)
    grid_spec = pltpu.PrefetchScalarGridSpec(
        num_scalar_prefetch=2,
        grid=(N // R,),
        in_specs=[
            pl.BlockSpec((R, D), lambda i, *_: (i, 0)),
            pl.BlockSpec((N, D), lambda i, *_: (0, 0)),
            pl.BlockSpec((R, 1), lambda i, *_: (i, 0)),
            pl.BlockSpec((1, N), lambda i, *_: (0, 0)),
            pl.BlockSpec((D, D), lambda i, *_: (0, 0)),
            pl.BlockSpec((1, D), lambda i, *_: (0, 0)),
            pl.BlockSpec((R, D), lambda i, *_: (i, 0)),
        ],
        out_specs=[
            pl.BlockSpec((R, D), lambda i, *_: (i, 0)),
            pl.BlockSpec((1, 1), lambda i, *_: (0, 0)),
        ],
        scratch_shapes=[pltpu.VMEM((R, N), jnp.int32)],
    )
    out_shape = [
        jax.ShapeDtypeStruct((N, D), jnp.float32),
        jax.ShapeDtypeStruct((1, 1), jnp.float32),
    ]
    return pl.pallas_call(kern, grid_spec=grid_spec, out_shape=out_shape)(
        lo_al, hi, h, h, brow, bcol, w, b, y)


def _pad_w(w):
    return jnp.pad(w, ((0, D - w.shape[0]), (0, D - w.shape[1])))


def _pad_b(b):
    return jnp.pad(b, (0, D - b.shape[0])).reshape(1, D)


def kernel(x, batch, y, W1, b1, W2, b2, W3, b3):
    h = jnp.pad(x, ((0, 0), (0, D - x.shape[1])))
    yp = jnp.pad(y, ((0, 0), (0, D - y.shape[1])))
    brow = batch.reshape(N, 1)
    bcol = batch.reshape(1, N)
    r0 = jnp.arange(0, N, R)
    b0 = batch[r0]
    b1_ = batch[r0 + R - 1]
    lo = jnp.searchsorted(batch, b0, side="left").astype(jnp.int32)
    hi = jnp.searchsorted(batch, b1_, side="right").astype(jnp.int32)
    lo_al = (lo // ALIGN) * ALIGN
    h1, _ = _layer(h, brow, bcol, lo_al, hi, _pad_w(W1), _pad_b(b1), yp,
                   True, False)
    h2, _ = _layer(h1, brow, bcol, lo_al, hi, _pad_w(W2), _pad_b(b2), yp,
                   True, False)
    h3, loss = _layer(h2, brow, bcol, lo_al, hi, _pad_w(W3), _pad_b(b3),
                      yp, False, True)
    return h3[:, :3], loss[0, 0]


# speedup vs baseline: 1.8739x; 1.0579x over previous
import functools

import jax
import jax.numpy as jnp
import numpy as np
from jax.experimental import pallas as pl
from jax.experimental.pallas import tpu as pltpu

N = 8192
K = 32
D = 128
R = 256
C = 1024
NCHUNK = N // C
WCHUNK = 3
ALIGN = 128
IMAX = np.int32(0x7FFFFFFF)
def _dot3(mb, a, dn):
    a1 = a.astype(jnp.bfloat16)
    r1 = a - a1.astype(jnp.float32)
    a2 = r1.astype(jnp.bfloat16)
    r2 = r1 - a2.astype(jnp.float32)
    a3 = r2.astype(jnp.bfloat16)

    def d(x):
        return jax.lax.dot_general(mb, x, dn,
                                   preferred_element_type=jnp.float32)

    return d(a1) + d(a2) + d(a3)
_PREC_REF = jax.lax.Precision.DEFAULT


def _phases(i, hr, sqr, br, ha_ref, sqn_ref, bcol_ref, keys_ref, w_ref,
            b_ref, y_ref, out_ref, loss_ref, lo, nchunk, csize, relu, last):

    for ci in range(nchunk):
        off = pl.multiple_of(lo + ci * csize, ALIGN)
        ha_c = ha_ref[pl.ds(off, csize), :]
        g = jax.lax.dot_general(hr, ha_c, (((1,), (1,)), ((), ())),
                                preferred_element_type=jnp.float32,
                                precision=_PREC_REF)
        sqc = sqn_ref[0:1, pl.ds(off, csize)]
        dist = sqr + sqc - 2.0 * g
        u = jax.lax.bitcast_convert_type(dist, jnp.int32)
        key = u ^ ((u >> 31) & IMAX)
        bc = bcol_ref[0:1, pl.ds(off, csize)]
        col_ids = off + jax.lax.broadcasted_iota(jnp.int32, (R, csize), 1)
        row_ids = i * R + jax.lax.broadcasted_iota(jnp.int32, (R, csize), 0)
        valid = (br == bc) & (col_ids != row_ids)
        keys_ref[:, ci * csize:(ci + 1) * csize] = jnp.where(valid, key, IMAX)

    def count_le(t):
        c = jnp.zeros((R, 1), jnp.int32)
        for ci in range(nchunk):
            kc = keys_ref[:, ci * csize:(ci + 1) * csize]
            c = c + jnp.sum((kc <= t).astype(jnp.int32), axis=1,
                            keepdims=True)
        return c

    mc = keys_ref[:, 0:csize]
    for ci in range(1, nchunk):
        mc = jnp.minimum(mc, keys_ref[:, ci * csize:(ci + 1) * csize])
    w = csize
    while w > 32:
        w //= 2
        mc = jnp.minimum(mc[:, :w], mc[:, w:2 * w])
    ub = jnp.max(mc, axis=1, keepdims=True)
    lb = jnp.min(mc, axis=1, keepdims=True)

    def bi_round(lo_, hi_, v_, res):
        d = hi_ - lo_
        mid = lo_ + ((d >> 1) & IMAX)
        c = count_le(mid)
        hit = jnp.logical_and(c == K, res == 0)
        v_ = jnp.where(hit, mid, v_)
        res = jnp.where(hit, jnp.int32(1), res)
        lt = c < K
        lo_ = jnp.where(lt, mid, lo_)
        hi_ = jnp.where(lt, hi_, mid)
        return lo_, hi_, v_, res

    def bi_cond(carry):
        it, _, _, _, res = carry
        return jnp.logical_and(it < 17, jnp.sum(res) < R)

    def bi_body(carry):
        it, lo_, hi_, v_, res = carry
        lo_, hi_, v_, res = bi_round(lo_, hi_, v_, res)
        lo_, hi_, v_, res = bi_round(lo_, hi_, v_, res)
        return it + 1, lo_, hi_, v_, res

    zero = jnp.zeros((R, 1), jnp.int32)
    _, _, hi_f, v, res_f = jax.lax.while_loop(
        bi_cond, bi_body, (jnp.int32(0), lb - 1, ub, zero, zero))
    v = jnp.where(res_f == 1, v, hi_f)

    acc = jnp.zeros((R, D), jnp.float32)
    cnt = jnp.zeros((R, 1), jnp.float32)
    for ci in range(nchunk):
        kc = keys_ref[:, ci * csize:(ci + 1) * csize]
        mc = ((kc <= v) & (kc != IMAX)).astype(jnp.float32)
        cnt = cnt + jnp.sum(mc, axis=1, keepdims=True)
        ha_c = ha_ref[pl.ds(pl.multiple_of(lo + ci * csize, ALIGN), csize), :]
        acc = acc + _dot3(mc.astype(jnp.bfloat16), ha_c,
                          (((1,), (0,)), ((), ())))
    agg = acc / jnp.maximum(cnt, 1.0)

    out = jax.lax.dot_general(agg, w_ref[...], (((1,), (0,)), ((), ())),
                              preferred_element_type=jnp.float32,
                              precision=_PREC_REF) + b_ref[...]
    if relu:
        out = jnp.maximum(out, 0.0)
    out_ref[...] = out

    if last:
        yb = y_ref[...]
        d2 = (out - yb) ** 2
        part = jnp.sum(jnp.sum(d2, axis=1, keepdims=True), axis=0,
                       keepdims=True)
        prev = jnp.where(i == 0, jnp.zeros((1, 1), jnp.float32),
                         loss_ref[...])
        total = prev + part
        loss_ref[...] = jnp.where(i == pl.num_programs(0) - 1,
                                  total / jnp.float32(N * 3), total)


def _layer_kernel(lo_ref, hi_ref, hr_ref, ha_ref, sqn_ref, brow_ref,
                  bcol_ref, w_ref, b_ref, y_ref, out_ref, loss_ref,
                  keys_ref, *, relu, last):
    i = pl.program_id(0)
    hr = hr_ref[...]
    sqr = jnp.sum(hr * hr, axis=1, keepdims=True)
    br = brow_ref[...]
    body = functools.partial(_phases, i, hr, sqr, br, ha_ref, sqn_ref,
                             bcol_ref, keys_ref, w_ref, b_ref, y_ref,
                             out_ref, loss_ref, relu=relu, last=last)
    lo_a = lo_ref[i]
    hi = hi_ref[i]
    lo1 = jnp.minimum(lo_a, jnp.int32(N - 1536))
    fits1 = hi - lo1 <= 1536
    lo3 = jnp.minimum(lo_a, jnp.int32(N - 3 * C))
    fits3 = hi - lo3 <= 3 * C

    @pl.when(fits1)
    def _win1():
        body(lo=lo1, nchunk=3, csize=512)

    @pl.when(jnp.logical_and(jnp.logical_not(fits1), fits3))
    def _win3():
        body(lo=lo3, nchunk=3, csize=C)

    @pl.when(jnp.logical_not(fits3))
    def _full():
        body(lo=jnp.int32(0), nchunk=NCHUNK, csize=C)


def _norms_kernel(hc_ref, out_ref):
    ones = jnp.ones((1, D), jnp.bfloat16)
    hc = hc_ref[...]
    out_ref[...] = _dot3(ones, hc * hc, (((1,), (1,)), ((), ())))


def _norms(h):
    return pl.pallas_call(
        _norms_kernel, grid=(NCHUNK,),
        in_specs=[pl.BlockSpec((C, D), lambda i: (i, 0))],
        out_specs=pl.BlockSpec((1, C), lambda i: (0, i)),
        out_shape=jax.ShapeDtypeStruct((1, N), jnp.float32),
    )(h)


def _layer(h, brow, bcol, lo_al, hi, w, b, y, relu, last):
    sqn = _norms(h)
    kern = functools.partial(_layer_kernel, relu=relu, last=last)
    grid_spec = pltpu.PrefetchScalarGridSpec(
        num_scalar_prefetch=2,
        grid=(N // R,),
        in_specs=[
            pl.BlockSpec((R, D), lambda i, *_: (i, 0)),
            pl.BlockSpec((N, D), lambda i, *_: (0, 0)),
            pl.BlockSpec((1, N), lambda i, *_: (0, 0)),
            pl.BlockSpec((R, 1), lambda i, *_: (i, 0)),
            pl.BlockSpec((1, N), lambda i, *_: (0, 0)),
            pl.BlockSpec((D, D), lambda i, *_: (0, 0)),
            pl.BlockSpec((1, D), lambda i, *_: (0, 0)),
            pl.BlockSpec((R, D), lambda i, *_: (i, 0)),
        ],
        out_specs=[
            pl.BlockSpec((R, D), lambda i, *_: (i, 0)),
            pl.BlockSpec((1, 1), lambda i, *_: (0, 0)),
        ],
        scratch_shapes=[pltpu.VMEM((R, N), jnp.int32)],
    )
    out_shape = [
        jax.ShapeDtypeStruct((N, D), jnp.float32),
        jax.ShapeDtypeStruct((1, 1), jnp.float32),
    ]
    return pl.pallas_call(kern, grid_spec=grid_spec, out_shape=out_shape)(
        lo_al, hi, h, h, sqn, brow, bcol, w, b, y)


def _pad_w(w):
    return jnp.pad(w, ((0, D - w.shape[0]), (0, D - w.shape[1])))


def _pad_b(b):
    return jnp.pad(b, (0, D - b.shape[0])).reshape(1, D)


def kernel(x, batch, y, W1, b1, W2, b2, W3, b3):
    h = jnp.pad(x, ((0, 0), (0, D - x.shape[1])))
    yp = jnp.pad(y, ((0, 0), (0, D - y.shape[1])))
    brow = batch.reshape(N, 1)
    bcol = batch.reshape(1, N)
    r0 = jnp.arange(0, N, R)
    b0 = batch[r0]
    b1_ = batch[r0 + R - 1]
    lo = jnp.searchsorted(batch, b0, side="left").astype(jnp.int32)
    hi = jnp.searchsorted(batch, b1_, side="right").astype(jnp.int32)
    lo_al = (lo // ALIGN) * ALIGN
    h1, _ = _layer(h, brow, bcol, lo_al, hi, _pad_w(W1), _pad_b(b1), yp,
                   True, False)
    h2, _ = _layer(h1, brow, bcol, lo_al, hi, _pad_w(W2), _pad_b(b2), yp,
                   True, False)
    h3, loss = _layer(h2, brow, bcol, lo_al, hi, _pad_w(W3), _pad_b(b3),
                      yp, False, True)
    return h3[:, :3], loss[0, 0]


# speedup vs baseline: 1.8741x; 1.0001x over previous
import functools

import jax
import jax.numpy as jnp
import numpy as np
from jax.experimental import pallas as pl
from jax.experimental.pallas import tpu as pltpu

N = 8192
K = 32
D = 128
R = 256
C = 1024
NCHUNK = N // C
ALIGN = 128
IMAX = np.int32(0x7FFFFFFF)
def _dot3(mb, a, dn):
    a1 = a.astype(jnp.bfloat16)
    r1 = a - a1.astype(jnp.float32)
    a2 = r1.astype(jnp.bfloat16)
    r2 = r1 - a2.astype(jnp.float32)
    a3 = r2.astype(jnp.bfloat16)

    def d(x):
        return jax.lax.dot_general(mb, x, dn,
                                   preferred_element_type=jnp.float32)

    return d(a1) + d(a2) + d(a3)
_PREC_REF = jax.lax.Precision.DEFAULT


def _phases(i, hr, sqr, br, ha_ref, sqn_ref, bcol_ref, keys_ref, w_ref,
            b_ref, y_ref, out_ref, loss_ref, lo, nchunk, csize, relu, last):

    for ci in range(nchunk):
        off = pl.multiple_of(lo + ci * csize, ALIGN)
        ha_c = ha_ref[pl.ds(off, csize), :]
        g = jax.lax.dot_general(hr, ha_c, (((1,), (1,)), ((), ())),
                                preferred_element_type=jnp.float32,
                                precision=_PREC_REF)
        sqc = sqn_ref[0:1, pl.ds(off, csize)]
        dist = sqr + sqc - 2.0 * g
        u = jax.lax.bitcast_convert_type(dist, jnp.int32)
        key = u ^ ((u >> 31) & IMAX)
        bc = bcol_ref[0:1, pl.ds(off, csize)]
        col_ids = off + jax.lax.broadcasted_iota(jnp.int32, (R, csize), 1)
        row_ids = i * R + jax.lax.broadcasted_iota(jnp.int32, (R, csize), 0)
        valid = (br == bc) & (col_ids != row_ids)
        keys_ref[:, ci * csize:(ci + 1) * csize] = jnp.where(valid, key, IMAX)

    def count_le(t):
        c = jnp.zeros((R, 1), jnp.int32)
        for ci in range(nchunk):
            kc = keys_ref[:, ci * csize:(ci + 1) * csize]
            c = c + jnp.sum((kc <= t).astype(jnp.int32), axis=1,
                            keepdims=True)
        return c

    mc = keys_ref[:, 0:csize]
    for ci in range(1, nchunk):
        mc = jnp.minimum(mc, keys_ref[:, ci * csize:(ci + 1) * csize])
    w = csize
    while w > 32:
        w //= 2
        mc = jnp.minimum(mc[:, :w], mc[:, w:2 * w])
    ub = jnp.max(mc, axis=1, keepdims=True)
    lb = jnp.min(mc, axis=1, keepdims=True)

    def bi_round(lo_, hi_, v_, res):
        d = hi_ - lo_
        mid = lo_ + ((d >> 1) & IMAX)
        c = count_le(mid)
        hit = jnp.logical_and(c == K, res == 0)
        v_ = jnp.where(hit, mid, v_)
        res = jnp.where(hit, jnp.int32(1), res)
        lt = c < K
        lo_ = jnp.where(lt, mid, lo_)
        hi_ = jnp.where(lt, hi_, mid)
        return lo_, hi_, v_, res

    def bi_cond(carry):
        it, _, _, _, res = carry
        return jnp.logical_and(it < 17, jnp.sum(res) < R)

    def bi_body(carry):
        it, lo_, hi_, v_, res = carry
        lo_, hi_, v_, res = bi_round(lo_, hi_, v_, res)
        lo_, hi_, v_, res = bi_round(lo_, hi_, v_, res)
        return it + 1, lo_, hi_, v_, res

    zero = jnp.zeros((R, 1), jnp.int32)
    _, _, hi_f, v, res_f = jax.lax.while_loop(
        bi_cond, bi_body, (jnp.int32(0), lb - 1, ub, zero, zero))
    v = jnp.where(res_f == 1, v, hi_f)

    acc = jnp.zeros((R, D), jnp.float32)
    cnt = jnp.zeros((R, 1), jnp.float32)
    for ci in range(nchunk):
        kc = keys_ref[:, ci * csize:(ci + 1) * csize]
        mc = ((kc <= v) & (kc != IMAX)).astype(jnp.float32)
        cnt = cnt + jnp.sum(mc, axis=1, keepdims=True)
        ha_c = ha_ref[pl.ds(pl.multiple_of(lo + ci * csize, ALIGN), csize), :]
        acc = acc + _dot3(mc.astype(jnp.bfloat16), ha_c,
                          (((1,), (0,)), ((), ())))
    agg = acc / jnp.maximum(cnt, 1.0)

    out = jax.lax.dot_general(agg, w_ref[...], (((1,), (0,)), ((), ())),
                              preferred_element_type=jnp.float32,
                              precision=_PREC_REF) + b_ref[...]
    if relu:
        out = jnp.maximum(out, 0.0)
    out_ref[...] = out

    if last:
        yb = y_ref[...]
        d2 = (out - yb) ** 2
        part = jnp.sum(jnp.sum(d2, axis=1, keepdims=True), axis=0,
                       keepdims=True)
        prev = jnp.where(i == 0, jnp.zeros((1, 1), jnp.float32),
                         loss_ref[...])
        total = prev + part
        loss_ref[...] = jnp.where(i == pl.num_programs(0) - 1,
                                  total / jnp.float32(N * 3), total)


def _layer_kernel(lo_ref, hi_ref, hr_ref, ha_ref, sqn_ref, brow_ref,
                  bcol_ref, w_ref, b_ref, y_ref, out_ref, loss_ref,
                  keys_ref, *, relu, last):
    i = pl.program_id(0)
    hr = hr_ref[...]
    sqr = jnp.sum(hr * hr, axis=1, keepdims=True)
    br = brow_ref[...]
    body = functools.partial(_phases, i, hr, sqr, br, ha_ref, sqn_ref,
                             bcol_ref, keys_ref, w_ref, b_ref, y_ref,
                             out_ref, loss_ref, relu=relu, last=last)
    lo_a = lo_ref[i]
    hi = hi_ref[i]
    lo1 = jnp.minimum(lo_a, jnp.int32(N - 1536))
    fits1 = hi - lo1 <= 1536
    lo3 = jnp.minimum(lo_a, jnp.int32(N - 3 * C))
    fits3 = hi - lo3 <= 3 * C

    @pl.when(fits1)
    def _win1():
        body(lo=lo1, nchunk=3, csize=512)

    @pl.when(jnp.logical_and(jnp.logical_not(fits1), fits3))
    def _win3():
        body(lo=lo3, nchunk=3, csize=C)

    @pl.when(jnp.logical_not(fits3))
    def _full():
        body(lo=jnp.int32(0), nchunk=NCHUNK, csize=C)


def _norms_kernel(hc_ref, out_ref):
    ones = jnp.ones((1, D), jnp.bfloat16)
    hc = hc_ref[...]
    out_ref[...] = _dot3(ones, hc * hc, (((1,), (1,)), ((), ())))


def _norms(h):
    return pl.pallas_call(
        _norms_kernel, grid=(NCHUNK,),
        in_specs=[pl.BlockSpec((C, D), lambda i: (i, 0))],
        out_specs=pl.BlockSpec((1, C), lambda i: (0, i)),
        out_shape=jax.ShapeDtypeStruct((1, N), jnp.float32),
    )(h)


def _layer(h, brow, bcol, lo_al, hi, w, b, y, relu, last):
    sqn = _norms(h)
    kern = functools.partial(_layer_kernel, relu=relu, last=last)
    grid_spec = pltpu.PrefetchScalarGridSpec(
        num_scalar_prefetch=2,
        grid=(N // R,),
        in_specs=[
            pl.BlockSpec((R, D), lambda i, *_: (i, 0)),
            pl.BlockSpec((N, D), lambda i, *_: (0, 0)),
            pl.BlockSpec((1, N), lambda i, *_: (0, 0)),
            pl.BlockSpec((R, 1), lambda i, *_: (i, 0)),
            pl.BlockSpec((1, N), lambda i, *_: (0, 0)),
            pl.BlockSpec((D, D), lambda i, *_: (0, 0)),
            pl.BlockSpec((1, D), lambda i, *_: (0, 0)),
            pl.BlockSpec((R, D), lambda i, *_: (i, 0)),
        ],
        out_specs=[
            pl.BlockSpec((R, D), lambda i, *_: (i, 0)),
            pl.BlockSpec((1, 1), lambda i, *_: (0, 0)),
        ],
        scratch_shapes=[pltpu.VMEM((R, N), jnp.int32)],
    )
    out_shape = [
        jax.ShapeDtypeStruct((N, D), jnp.float32),
        jax.ShapeDtypeStruct((1, 1), jnp.float32),
    ]
    return pl.pallas_call(kern, grid_spec=grid_spec, out_shape=out_shape)(
        lo_al, hi, h, h, sqn, brow, bcol, w, b, y)


def _pad_w(w):
    return jnp.pad(w, ((0, D - w.shape[0]), (0, D - w.shape[1])))


def _pad_b(b):
    return jnp.pad(b, (0, D - b.shape[0])).reshape(1, D)


def kernel(x, batch, y, W1, b1, W2, b2, W3, b3):
    h = jnp.pad(x, ((0, 0), (0, D - x.shape[1])))
    yp = jnp.pad(y, ((0, 0), (0, D - y.shape[1])))
    brow = batch.reshape(N, 1)
    bcol = batch.reshape(1, N)
    r0 = jnp.arange(0, N, R)
    b0 = batch[r0]
    b1_ = batch[r0 + R - 1]
    lo = jnp.searchsorted(batch, b0, side="left").astype(jnp.int32)
    hi = jnp.searchsorted(batch, b1_, side="right").astype(jnp.int32)
    lo_al = (lo // ALIGN) * ALIGN
    h1, _ = _layer(h, brow, bcol, lo_al, hi, _pad_w(W1), _pad_b(b1), yp,
                   True, False)
    h2, _ = _layer(h1, brow, bcol, lo_al, hi, _pad_w(W2), _pad_b(b2), yp,
                   True, False)
    h3, loss = _layer(h2, brow, bcol, lo_al, hi, _pad_w(W3), _pad_b(b3),
                      yp, False, True)
    return h3[:, :3], loss[0, 0]
